# baseline (speedup 1.0000x reference)
import functools

import jax
import jax.numpy as jnp
from jax import lax
from jax.experimental import pallas as pl
from jax.experimental.pallas import tpu as pltpu

_K = 0.5
_INT_MIN = -(2 ** 31)
_POS_MASK = 2 ** 31 - 1


def _keys_from_scores(s):
    b = lax.bitcast_convert_type(s, jnp.int32)
    return jnp.where(b >= 0, b, b ^ _POS_MASK)


def _fused_kernel(s_ref, w_ref, x_ref, o_ref, inv_ref, twm_ref, *, j, rows,
                  blocks_per_c):
    step = pl.program_id(0) * pl.num_programs(1) + pl.program_id(1)

    @pl.when(step == 0)
    def _prologue():
        keys = _keys_from_scores(s_ref[...])

        def body(i, res_u):
            bit = lax.shift_left(jnp.int32(1), jnp.int32(31 - i))
            cand_u = res_u | bit
            cand_key = cand_u ^ jnp.int32(_INT_MIN)
            cnt = jnp.sum((keys < cand_key).astype(jnp.int32))
            return jnp.where(cnt <= j, cand_u, res_u)

        res_u = lax.fori_loop(0, 32, body, jnp.int32(0), unroll=True)
        t = res_u ^ jnp.int32(_INT_MIN)
        below = keys < t
        inv_ref[...] = below.astype(jnp.float32)
        twm_ref[...] = jnp.where(below, 0.0, jnp.tanh(w_ref[...]))

    base = step * rows
    inv = inv_ref[pl.ds(base, rows), :]
    twm = twm_ref[pl.ds(base, rows), :]
    o_ref[...] = x_ref[...] * inv[None, None] + twm[None, None]


@jax.jit
def kernel(x, scores, weight):
    n = scores.size
    j = int((1.0 - _K) * n)
    batch = x.shape[0]
    c, h, w = scores.shape

    rows = 64
    blocks_per_c = h // rows
    grid = (c, blocks_per_c)
    sf = scores.reshape(c * h, w)
    wf = weight.reshape(c * h, w)

    out = pl.pallas_call(
        functools.partial(_fused_kernel, j=j, rows=rows,
                          blocks_per_c=blocks_per_c),
        grid=grid,
        out_shape=jax.ShapeDtypeStruct(x.shape, jnp.float32),
        in_specs=[
            pl.BlockSpec((c * h, w), lambda ci, hi: (0, 0)),
            pl.BlockSpec((c * h, w), lambda ci, hi: (0, 0)),
            pl.BlockSpec((batch, 1, rows, w), lambda ci, hi: (0, ci, hi, 0)),
        ],
        out_specs=pl.BlockSpec((batch, 1, rows, w), lambda ci, hi: (0, ci, hi, 0)),
        scratch_shapes=[
            pltpu.VMEM((c * h, w), jnp.float32),
            pltpu.VMEM((c * h, w), jnp.float32),
        ],
        compiler_params=pltpu.CompilerParams(
            dimension_semantics=("arbitrary", "arbitrary"),
        ),
    )(sf, wf, x)
    return out
